# baseline (speedup 1.0000x reference)
import functools

import jax
import jax.numpy as jnp
from jax import lax
from jax.experimental import pallas as pl
from jax.experimental.pallas import tpu as pltpu
from jax.experimental.pallas import tpu_sc as plsc

B = 32
S = 512
H = 768
L = 16
NSL = H // L
NW = 32
SW = S // NW
EPS = 1e-12


def _rsqrt16(x):
    xi = lax.bitcast_convert_type(x, jnp.int32)
    yi = jnp.int32(0x5F3759DF) - lax.shift_right_logical(xi, 1)
    y = lax.bitcast_convert_type(yi, jnp.float32)
    xh = x * jnp.float32(0.5)
    for _ in range(3):
        y = y * (jnp.float32(1.5) - xh * y * y)
    return y


_GDN = lax.GatherDimensionNumbers(
    offset_dims=(), collapsed_slice_dims=(0,), start_index_map=(0,))


def _shuffle16(x, idx):
    return lax.gather(x, idx[:, None], _GDN, slice_sizes=(1,),
                      mode=lax.GatherScatterMode.PROMISE_IN_BOUNDS)


def _hsum16(x):
    i = lax.iota(jnp.int32, L)
    for sh in (8, 4, 2, 1):
        x = x + _shuffle16(x, (i + sh) & (L - 1))
    return x


def _compute_chunk(row_buf, pos_v, gam_v, bet_v):
    def tok(t, carry):
        row = row_buf.at[t]
        prow = pos_v.at[t]
        s = [jnp.zeros((L,), jnp.float32) for _ in range(4)]
        q = [jnp.zeros((L,), jnp.float32) for _ in range(4)]
        for i in range(NSL):
            v = row[pl.ds(L * i, L)] + prow[pl.ds(L * i, L)]
            row[pl.ds(L * i, L)] = v
            s[i % 4] = s[i % 4] + v
            q[i % 4] = q[i % 4] + v * v
        ssum = (s[0] + s[1]) + (s[2] + s[3])
        qsum = (q[0] + q[1]) + (q[2] + q[3])
        mv = _hsum16(ssum) * jnp.float32(1.0 / H)
        var = _hsum16(qsum) * jnp.float32(1.0 / H) - mv * mv
        rstd = _rsqrt16(var + jnp.float32(EPS))
        for i in range(NSL):
            xv = row[pl.ds(L * i, L)]
            o = (xv - mv) * rstd
            o = o * gam_v[pl.ds(L * i, L)] + bet_v[pl.ds(L * i, L)]
            row[pl.ds(L * i, L)] = o
        return carry

    lax.fori_loop(0, SW, tok, 0)


def _emb_body(ids_h, word_h, pos_h, gam_h, bet_h, out_h,
              idx_v, pos_v, gam_v, bet_v, buf, sem0, sem1):
    cid = lax.axis_index("c")
    sid = lax.axis_index("s")
    w = sid * 2 + cid
    s_base = pl.multiple_of(w * SW, SW)

    pltpu.sync_copy(ids_h, idx_v)
    pltpu.sync_copy(pos_h.at[pl.ds(s_base, SW)], pos_v)
    pltpu.sync_copy(gam_h, gam_v)
    pltpu.sync_copy(bet_h, bet_v)

    def idx_ref(b):
        return idx_v.at[pl.ds(pl.multiple_of(b * S + s_base, SW), SW)]

    sems = (sem0, sem1)
    pltpu.async_copy(word_h.at[idx_ref(0)], buf.at[0], sem0)
    pltpu.async_copy(word_h.at[idx_ref(1)], buf.at[1], sem1)

    def outer(j, carry):
        for k in range(2):
            b = 2 * j + k
            pltpu.make_async_copy(
                word_h.at[idx_ref(b)], buf.at[k], sems[k]).wait()
            _compute_chunk(buf.at[k], pos_v, gam_v, bet_v)
            pltpu.sync_copy(buf.at[k], out_h.at[b, pl.ds(s_base, SW)])

            @pl.when(b + 2 < B)
            def _():
                pltpu.async_copy(word_h.at[idx_ref(b + 2)], buf.at[k], sems[k])
        return carry

    lax.fori_loop(0, B // 2, outer, 0)


@functools.partial(jax.jit, static_argnums=())
def _emb_call(ids, word_table, pos_table, ln_gamma, ln_beta):
    mesh = plsc.VectorSubcoreMesh(core_axis_name="c", subcore_axis_name="s")
    f = pl.kernel(
        _emb_body,
        mesh=mesh,
        out_type=jax.ShapeDtypeStruct((B, S, H), jnp.float32),
        scratch_types=[
            pltpu.VMEM((B * S,), jnp.int32),
            pltpu.VMEM((SW, H), jnp.float32),
            pltpu.VMEM((H,), jnp.float32),
            pltpu.VMEM((H,), jnp.float32),
            pltpu.VMEM((2, SW, H), jnp.float32),
            pltpu.SemaphoreType.DMA,
            pltpu.SemaphoreType.DMA,
        ],
    )
    return f(ids, word_table, pos_table, ln_gamma, ln_beta)


def kernel(input_ids, word_table, pos_table, ln_gamma, ln_beta):
    ids_flat = input_ids.astype(jnp.int32).reshape(B * S)
    return _emb_call(ids_flat, word_table, pos_table, ln_gamma, ln_beta)


# speedup vs baseline: 2.2978x; 2.2978x over previous
import functools

import jax
import jax.numpy as jnp
from jax import lax
from jax.experimental import pallas as pl
from jax.experimental.pallas import tpu as pltpu
from jax.experimental.pallas import tpu_sc as plsc

B = 32
S = 512
H = 768
NW = 32
C = 64
NCH = S // C
EPS = 1e-12


def _gather_body(ids_h, word_h, tmp_h, idx_v, buf, sem0, sem1):
    cid = lax.axis_index("c")
    sid = lax.axis_index("s")
    w = sid * 2 + cid
    base = pl.multiple_of(w * S, S)

    pltpu.sync_copy(ids_h.at[pl.ds(base, S)], idx_v)

    sems = (sem0, sem1)

    def gather(c, k):
        pltpu.async_copy(
            word_h.at[idx_v.at[pl.ds(c * C, C)]], buf.at[k], sems[k])

    gather(0, 0)
    gather(1, 1)
    for c in range(NCH):
        k = c % 2
        pltpu.make_async_copy(
            word_h.at[idx_v.at[pl.ds(c * C, C)]], buf.at[k], sems[k]).wait()
        pltpu.sync_copy(buf.at[k], tmp_h.at[pl.ds(base + c * C, C)])
        if c + 2 < NCH:
            gather(c + 2, k)


def _sc_gather(ids_flat, word_table):
    mesh = plsc.VectorSubcoreMesh(core_axis_name="c", subcore_axis_name="s")
    f = pl.kernel(
        _gather_body,
        mesh=mesh,
        out_type=jax.ShapeDtypeStruct((B * S, H), jnp.float32),
        scratch_types=[
            pltpu.VMEM((S,), jnp.int32),
            pltpu.VMEM((2, C, H), jnp.float32),
            pltpu.SemaphoreType.DMA,
            pltpu.SemaphoreType.DMA,
        ],
    )
    return f(ids_flat, word_table)


def _ln_body(tmp_ref, pos_ref, gam_ref, bet_ref, out_ref):
    x = tmp_ref[...] + pos_ref[...]
    mean = jnp.mean(x, axis=-1, keepdims=True)
    xc = x - mean
    var = jnp.mean(xc * xc, axis=-1, keepdims=True)
    out_ref[...] = xc * lax.rsqrt(var + EPS) * gam_ref[...] + bet_ref[...]


def _tc_layernorm(tmp, pos_table, ln_gamma, ln_beta):
    return pl.pallas_call(
        _ln_body,
        grid=(B,),
        in_specs=[
            pl.BlockSpec((S, H), lambda b: (b, 0)),
            pl.BlockSpec((S, H), lambda b: (0, 0)),
            pl.BlockSpec((1, H), lambda b: (0, 0)),
            pl.BlockSpec((1, H), lambda b: (0, 0)),
        ],
        out_specs=pl.BlockSpec((S, H), lambda b: (b, 0)),
        out_shape=jax.ShapeDtypeStruct((B * S, H), jnp.float32),
    )(tmp, pos_table, ln_gamma.reshape(1, H), ln_beta.reshape(1, H))


@jax.jit
def _emb_call(ids_flat, word_table, pos_table, ln_gamma, ln_beta):
    tmp = _sc_gather(ids_flat, word_table)
    out = _tc_layernorm(tmp, pos_table, ln_gamma, ln_beta)
    return out.reshape(B, S, H)


def kernel(input_ids, word_table, pos_table, ln_gamma, ln_beta):
    ids_flat = input_ids.astype(jnp.int32).reshape(B * S)
    return _emb_call(ids_flat, word_table, pos_table, ln_gamma, ln_beta)
